# speedup vs baseline: 1.0143x
import functools
import jax
import jax.numpy as jnp
from jax import lax
from jax.experimental import pallas as pl
from jax.experimental.pallas import tpu as pltpu

D = 128
R_BLK = 512


def _proj_body(x_ref, w_ref, b_ref, o_ref):
    o_ref[...] = lax.dot_general(
        x_ref[...], w_ref[...], (((1,), (1,)), ((), ())),
        preferred_element_type=jnp.float32) + b_ref[...]


def _proj(x, W, b):
    n = x.shape[0]
    grid = (n + R_BLK - 1) // R_BLK
    return pl.pallas_call(
        _proj_body,
        grid=(grid,),
        in_specs=[
            pl.BlockSpec((R_BLK, D), lambda i: (i, 0)),
            pl.BlockSpec((D, D), lambda i: (0, 0)),
            pl.BlockSpec((1, D), lambda i: (0, 0)),
        ],
        out_specs=pl.BlockSpec((R_BLK, D), lambda i: (i, 0)),
        out_shape=jax.ShapeDtypeStruct((n, D), jnp.float32),
    )(x, W, b.reshape(1, D))


def _sage_body(s_ref, cnt_ref, x_ref, wl_ref, bl_ref, wr_ref, o_ref):
    c = cnt_ref[0, :] + cnt_ref[1, :]
    inv = 1.0 / jnp.maximum(c, 1.0)
    mean = s_ref[...] * inv[:, None]
    o = (lax.dot_general(mean, wl_ref[...], (((1,), (1,)), ((), ())),
                         preferred_element_type=jnp.float32)
         + bl_ref[...]
         + lax.dot_general(x_ref[...], wr_ref[...], (((1,), (1,)), ((), ())),
                           preferred_element_type=jnp.float32))
    o_ref[...] = jnp.maximum(o, 0.0)


def _sage_dense(s, cnt2, x, Wl, bl, Wr):
    n = s.shape[0]
    grid = (n + R_BLK - 1) // R_BLK
    return pl.pallas_call(
        _sage_body,
        grid=(grid,),
        in_specs=[
            pl.BlockSpec((R_BLK, D), lambda i: (i, 0)),
            pl.BlockSpec((2, R_BLK), lambda i: (0, i)),
            pl.BlockSpec((R_BLK, D), lambda i: (i, 0)),
            pl.BlockSpec((D, D), lambda i: (0, 0)),
            pl.BlockSpec((1, D), lambda i: (0, 0)),
            pl.BlockSpec((D, D), lambda i: (0, 0)),
        ],
        out_specs=pl.BlockSpec((R_BLK, D), lambda i: (i, 0)),
        out_shape=jax.ShapeDtypeStruct((n, D), jnp.float32),
    )(s, cnt2, x, Wl, bl.reshape(1, D), Wr)


def _npad(n):
    m = 16 * R_BLK
    return ((n + m - 1) // m) * m


def _segsum(table, src, dst, n_dst):
    s = jax.ops.segment_sum(table[src], dst, num_segments=n_dst)
    cnt = jax.ops.segment_sum(jnp.ones((src.shape[0],), jnp.float32), dst,
                              num_segments=n_dst)
    np_ = _npad(n_dst)
    cnt2 = jnp.zeros((2, np_), jnp.float32).at[0, :n_dst].set(cnt)
    return s, cnt2


def kernel(venue_x, edge_uv_src, edge_uv_dst, edge_vu_src, edge_vu_dst,
           emb_user, Wp, bp,
           Wl_uv_0, bl_uv_0, Wr_uv_0, Wl_vu_0, bl_vu_0, Wr_vu_0,
           Wl_uv_1, bl_uv_1, Wr_uv_1, Wl_vu_1, bl_vu_1, Wr_vu_1):
    n_user = emb_user.shape[0]
    n_venue = venue_x.shape[0]

    user = emb_user
    venue = _proj(venue_x, Wp, bp)

    layers = [
        (Wl_uv_0, bl_uv_0, Wr_uv_0, Wl_vu_0, bl_vu_0, Wr_vu_0),
        (Wl_uv_1, bl_uv_1, Wr_uv_1, Wl_vu_1, bl_vu_1, Wr_vu_1),
    ]
    for (Wluv, bluv, Wruv, Wlvu, blvu, Wrvu) in layers:
        s_v, cnt_v = _segsum(user, edge_uv_src, edge_uv_dst, n_venue)
        s_u, cnt_u = _segsum(venue, edge_vu_src, edge_vu_dst, n_user)
        venue_new = _sage_dense(s_v, cnt_v, venue, Wluv, bluv, Wruv)
        user_new = _sage_dense(s_u, cnt_u, user, Wlvu, blvu, Wrvu)
        user, venue = user_new, venue_new
    return (user, venue)


# speedup vs baseline: 1.8994x; 1.8994x over previous
import functools
import jax
import jax.numpy as jnp
from jax import lax
from jax.experimental import pallas as pl
from jax.experimental.pallas import tpu as pltpu
from jax.experimental.pallas import tpu_sc as plsc

D = 128
R_BLK = 512
NC = 2
NS = 16
ECHUNK = 128
OCHUNK = 125


def _pad_up(n, m):
    return ((n + m - 1) // m) * m


@functools.lru_cache(maxsize=None)
def _make_segsum(n_src, n_dst, e_pad, cg):
    w = D // cg
    n_acc = _pad_up(n_dst + 1, NS * 128)
    rows_per_tile = n_dst // NS
    assert rows_per_tile % OCHUNK == 0
    n_out_chunks = rows_per_tile // OCHUNK
    zch = (n_acc // NS) // 128
    n_edge_chunks = e_pad // NS // ECHUNK
    passes = cg // NC
    mesh = plsc.VectorSubcoreMesh(core_axis_name="c", subcore_axis_name="s", num_cores=NC, num_subcores=NS)

    def body(tflat, src2, dst2, zeros_hbm, out,
             acc, sbuf, gbuf, dbuf, rows_b, zbuf, obuf):
        core = lax.axis_index("c")
        sub = lax.axis_index("s")
        pltpu.sync_copy(zeros_hbm, zbuf)
        crow0 = sub * (e_pad // NS // ECHUNK)
        zrow0 = sub * (n_acc // NS)
        orow0 = sub * rows_per_tile
        for p in range(passes):
            cg_id = core * passes + p

            def zstep(z, _):
                pltpu.sync_copy(zbuf, acc.at[pl.ds(zrow0 + 128 * z, 128)])
                return 0
            lax.fori_loop(0, zch, zstep, 0)
            plsc.subcore_barrier()

            def estep(j, _):
                pltpu.sync_copy(src2.at[crow0 + j], sbuf)
                pltpu.sync_copy(dst2.at[crow0 + j], dbuf)
                for k in range(8):
                    v = sbuf[pl.ds(16 * k, 16)]
                    gbuf[pl.ds(16 * k, 16)] = v * cg + cg_id
                pltpu.sync_copy(tflat.at[gbuf], rows_b)
                pltpu.sync_copy(rows_b, acc.at[dbuf], add=True)
                return 0
            lax.fori_loop(0, n_edge_chunks, estep, 0)
            plsc.subcore_barrier()

            def ostep(t, _):
                r0 = orow0 + OCHUNK * t
                pltpu.sync_copy(acc.at[pl.ds(r0, OCHUNK)], obuf)
                pltpu.sync_copy(obuf,
                                out.at[pl.ds(r0, OCHUNK), pl.ds(cg_id * w, w)])
                return 0
            lax.fori_loop(0, n_out_chunks, ostep, 0)
            if p + 1 < passes:
                plsc.subcore_barrier()

    return pl.kernel(
        body,
        out_type=jax.ShapeDtypeStruct((n_dst, D), jnp.float32),
        mesh=mesh,
        compiler_params=pltpu.CompilerParams(use_tc_tiling_on_sc=False),
        scratch_types=[
            pltpu.VMEM_SHARED((n_acc, w), jnp.float32),
            pltpu.VMEM((ECHUNK,), jnp.int32),
            pltpu.VMEM((ECHUNK,), jnp.int32),
            pltpu.VMEM((ECHUNK,), jnp.int32),
            pltpu.VMEM((ECHUNK, w), jnp.float32),
            pltpu.VMEM((128, w), jnp.float32),
            pltpu.VMEM((OCHUNK, w), jnp.float32),
        ],
    )


def _segsum_sc(table, src2, dst2, n_dst, cg):
    n_src = table.shape[0]
    w = D // cg
    tflat = table.reshape(n_src * cg, w)
    e_pad = src2.shape[0] * ECHUNK
    zeros = jnp.zeros((128, w), jnp.float32)
    kern = _make_segsum(n_src, n_dst, e_pad, cg)
    return kern(tflat, src2, dst2, zeros)


@functools.lru_cache(maxsize=None)
def _make_counts(n_dst, e_pad):
    n_acc = _pad_up(n_dst + 1, NS * 128)
    zch = (n_acc // NS) // 128
    n_edge_chunks = e_pad // (NC * NS) // ECHUNK
    mesh = plsc.VectorSubcoreMesh(core_axis_name="c", subcore_axis_name="s", num_cores=NC, num_subcores=NS)

    def body(dst2, ones_hbm, zeros_hbm, out, cnt_sh, dbuf, onesb, zbuf, tbuf):
        core = lax.axis_index("c")
        sub = lax.axis_index("s")
        pltpu.sync_copy(ones_hbm, onesb)
        pltpu.sync_copy(zeros_hbm, zbuf)
        zrow0 = sub * (n_acc // NS)

        def zstep(z, _):
            pltpu.sync_copy(zbuf, cnt_sh.at[pl.ds(zrow0 + 128 * z, 128)])
            return 0
        lax.fori_loop(0, zch, zstep, 0)
        plsc.subcore_barrier()

        wid = core * NS + sub
        crow0 = wid * n_edge_chunks

        def estep(j, _):
            pltpu.sync_copy(dst2.at[crow0 + j], dbuf)
            pltpu.sync_copy(onesb, cnt_sh.at[dbuf], add=True)
            return 0
        lax.fori_loop(0, n_edge_chunks, estep, 0)
        plsc.subcore_barrier()

        def ostep(z, _):
            r0 = zrow0 + 128 * z
            pltpu.sync_copy(cnt_sh.at[pl.ds(r0, 128)], tbuf)
            pltpu.sync_copy(tbuf, out.at[core, pl.ds(r0, 128)])
            return 0
        lax.fori_loop(0, zch, ostep, 0)

    return pl.kernel(
        body,
        out_type=jax.ShapeDtypeStruct((NC, n_acc, 16), jnp.float32),
        mesh=mesh,
        compiler_params=pltpu.CompilerParams(use_tc_tiling_on_sc=False),
        scratch_types=[
            pltpu.VMEM_SHARED((n_acc, 16), jnp.float32),
            pltpu.VMEM((ECHUNK,), jnp.int32),
            pltpu.VMEM((ECHUNK, 16), jnp.float32),
            pltpu.VMEM((128, 16), jnp.float32),
            pltpu.VMEM((128, 16), jnp.float32),
        ],
    )


def _counts_sc(dst2, n_dst):
    e_pad = dst2.shape[0] * ECHUNK
    ones = jnp.ones((ECHUNK, 16), jnp.float32)
    zeros = jnp.zeros((128, 16), jnp.float32)
    kern = _make_counts(n_dst, e_pad)
    out = kern(dst2, ones, zeros)
    return out[:, :, 0]


def _proj_body(x_ref, w_ref, b_ref, o_ref):
    o_ref[...] = lax.dot_general(
        x_ref[...], w_ref[...], (((1,), (1,)), ((), ())),
        preferred_element_type=jnp.float32) + b_ref[...]


def _proj(x, W, b):
    n = x.shape[0]
    grid = (n + R_BLK - 1) // R_BLK
    return pl.pallas_call(
        _proj_body,
        grid=(grid,),
        in_specs=[
            pl.BlockSpec((R_BLK, D), lambda i: (i, 0)),
            pl.BlockSpec((D, D), lambda i: (0, 0)),
            pl.BlockSpec((1, D), lambda i: (0, 0)),
        ],
        out_specs=pl.BlockSpec((R_BLK, D), lambda i: (i, 0)),
        out_shape=jax.ShapeDtypeStruct((n, D), jnp.float32),
    )(x, W, b.reshape(1, D))


def _sage_body(s_ref, cnt_ref, x_ref, wl_ref, bl_ref, wr_ref, o_ref):
    c = cnt_ref[0, :] + cnt_ref[1, :]
    inv = 1.0 / jnp.maximum(c, 1.0)
    mean = s_ref[...] * inv[:, None]
    o = (lax.dot_general(mean, wl_ref[...], (((1,), (1,)), ((), ())),
                         preferred_element_type=jnp.float32)
         + bl_ref[...]
         + lax.dot_general(x_ref[...], wr_ref[...], (((1,), (1,)), ((), ())),
                           preferred_element_type=jnp.float32))
    o_ref[...] = jnp.maximum(o, 0.0)


def _sage_dense(s, cnt2, x, Wl, bl, Wr):
    n = s.shape[0]
    grid = (n + R_BLK - 1) // R_BLK
    assert cnt2.shape[1] >= grid * R_BLK
    return pl.pallas_call(
        _sage_body,
        grid=(grid,),
        in_specs=[
            pl.BlockSpec((R_BLK, D), lambda i: (i, 0)),
            pl.BlockSpec((2, R_BLK), lambda i: (0, i)),
            pl.BlockSpec((R_BLK, D), lambda i: (i, 0)),
            pl.BlockSpec((D, D), lambda i: (0, 0)),
            pl.BlockSpec((1, D), lambda i: (0, 0)),
            pl.BlockSpec((D, D), lambda i: (0, 0)),
        ],
        out_specs=pl.BlockSpec((R_BLK, D), lambda i: (i, 0)),
        out_shape=jax.ShapeDtypeStruct((n, D), jnp.float32),
    )(s, cnt2, x, Wl, bl.reshape(1, D), Wr)


def _pad_edges(src, dst, n_dst):
    e = src.shape[0]
    e_pad = _pad_up(e, NC * NS * ECHUNK)
    pad = e_pad - e
    src_p = jnp.concatenate([src, jnp.zeros((pad,), jnp.int32)])
    dst_p = jnp.concatenate([dst, jnp.full((pad,), n_dst, jnp.int32)])
    return (src_p.reshape(e_pad // ECHUNK, ECHUNK),
            dst_p.reshape(e_pad // ECHUNK, ECHUNK))


def kernel(venue_x, edge_uv_src, edge_uv_dst, edge_vu_src, edge_vu_dst,
           emb_user, Wp, bp,
           Wl_uv_0, bl_uv_0, Wr_uv_0, Wl_vu_0, bl_vu_0, Wr_vu_0,
           Wl_uv_1, bl_uv_1, Wr_uv_1, Wl_vu_1, bl_vu_1, Wr_vu_1):
    n_user = emb_user.shape[0]
    n_venue = venue_x.shape[0]

    uv_src2, uv_dst2 = _pad_edges(edge_uv_src, edge_uv_dst, n_venue)
    vu_src2, vu_dst2 = _pad_edges(edge_vu_src, edge_vu_dst, n_user)

    cnt_v = _counts_sc(uv_dst2, n_venue)
    cnt_u = _counts_sc(vu_dst2, n_user)

    user = emb_user
    venue = _proj(venue_x, Wp, bp)

    layers = [
        (Wl_uv_0, bl_uv_0, Wr_uv_0, Wl_vu_0, bl_vu_0, Wr_vu_0),
        (Wl_uv_1, bl_uv_1, Wr_uv_1, Wl_vu_1, bl_vu_1, Wr_vu_1),
    ]
    for (Wluv, bluv, Wruv, Wlvu, blvu, Wrvu) in layers:
        s_v = _segsum_sc(user, uv_src2, uv_dst2, n_venue, cg=2)
        s_u = _segsum_sc(venue, vu_src2, vu_dst2, n_user, cg=4)
        venue_new = _sage_dense(s_v, cnt_v, venue, Wluv, bluv, Wruv)
        user_new = _sage_dense(s_u, cnt_u, user, Wlvu, blvu, Wrvu)
        user, venue = user_new, venue_new
    return (user, venue)


# speedup vs baseline: 3.8859x; 2.0459x over previous
import functools
import jax
import jax.numpy as jnp
from jax import lax
from jax.experimental import pallas as pl
from jax.experimental.pallas import tpu as pltpu
from jax.experimental.pallas import tpu_sc as plsc

D = 128
R_BLK = 512
NC = 2
NS = 16
ECHUNK = 128
NSLOT = 8
HLEAD = 4


def _pad_up(n, m):
    return ((n + m - 1) // m) * m


def _vcopy_idx(dst_ref, src_ref, j, cg, cg_id):
    for k in range(8):
        v = src_ref[j, pl.ds(16 * k, 16)]
        dst_ref[pl.ds(16 * k, 16)] = v * cg + cg_id


def _vcopy_row(dst_ref, src_ref, j):
    for k in range(8):
        dst_ref[pl.ds(16 * k, 16)] = src_ref[j, pl.ds(16 * k, 16)]


@functools.lru_cache(maxsize=None)
def _make_segsum(n_src, n_dst, e_pad, cg):
    w = D // cg
    nslot = 8 if w >= 64 else 6
    l1 = nslot - 2
    l2 = 2
    n_acc = _pad_up(n_dst + 1, NS * 128)
    zch = (n_acc // NS) // 128
    n = e_pad // NS // ECHUNK
    passes = cg // NC
    mesh = plsc.VectorSubcoreMesh(core_axis_name="c", subcore_axis_name="s",
                                  num_cores=NC, num_subcores=NS)

    def body(tflat, src2, dst2, zeros_hbm, out, *scr):
        acc = scr[0]
        sidx = scr[1:1 + nslot]
        dbuf = scr[1 + nslot:1 + 2 * nslot]
        rows = scr[1 + 2 * nslot:1 + 3 * nslot]
        isem = scr[1 + 3 * nslot:1 + 4 * nslot]
        gsem = scr[1 + 4 * nslot:1 + 5 * nslot]
        ssem = scr[1 + 5 * nslot:1 + 6 * nslot]
        osem = scr[1 + 6 * nslot]

        core = lax.axis_index("c")
        sub = lax.axis_index("s")
        crow0 = sub * n
        zrow0 = sub * (n_acc // NS)

        def fire_i(b, c):
            pltpu.async_copy(src2.at[crow0 + c], sidx[b], isem[b])
            pltpu.async_copy(dst2.at[crow0 + c], dbuf[b], isem[b])

        def wait_i(b, c):
            pltpu.make_async_copy(src2.at[crow0 + c], sidx[b], isem[b]).wait()
            pltpu.make_async_copy(dst2.at[crow0 + c], dbuf[b], isem[b]).wait()

        def fire_g(b, cg_id):
            for k in range(8):
                v = sidx[b][pl.ds(16 * k, 16)]
                sidx[b][pl.ds(16 * k, 16)] = v * cg + cg_id
            pltpu.async_copy(tflat.at[sidx[b]], rows[b], gsem[b])

        def wait_g(b):
            pltpu.make_async_copy(tflat.at[sidx[b]], rows[b], gsem[b]).wait()

        def fire_s(b):
            pltpu.async_copy(rows[b], acc.at[dbuf[b]], ssem[b], add=True)

        def wait_s(b):
            pltpu.make_async_copy(rows[b], acc.at[dbuf[b]], ssem[b]).wait()

        for p in range(passes):
            cg_id = core * passes + p

            pltpu.sync_copy(zeros_hbm, rows[0])
            for z in range(zch):
                pltpu.async_copy(
                    rows[0], acc.at[pl.ds(zrow0 + 128 * z, 128)], osem)
            for z in range(zch):
                pltpu.make_async_copy(
                    rows[0], acc.at[pl.ds(zrow0 + 128 * z, 128)], osem).wait()
            plsc.subcore_barrier()

            def one_iter(j, bj, b2, b1, do_g, do_i, ws):
                wait_g(bj)
                fire_s(bj)
                if do_g:
                    wait_i(b2, j + l2)
                    fire_g(b2, cg_id)
                if do_i:
                    if ws:
                        wait_s(b1)
                    fire_i(b1, j + l1)

            for c in range(l1):
                fire_i(c % nslot, c)
            for c in range(l2):
                wait_i(c % nslot, c)
                fire_g(c % nslot, cg_id)

            jm0 = nslot - l1
            jm1 = n - l1
            n_main = (jm1 - jm0) // nslot
            for j in range(jm0):
                one_iter(j, j % nslot, (j + l2) % nslot, (j + l1) % nslot,
                         True, True, False)

            def mstep(g, _):
                for b in range(nslot):
                    j = jm0 + g * nslot + b
                    one_iter(j, (jm0 + b) % nslot, (jm0 + b + l2) % nslot,
                             (jm0 + b + l1) % nslot, True, True, True)
                return 0
            lax.fori_loop(0, n_main, mstep, 0)

            for j in range(jm0 + n_main * nslot, n):
                one_iter(j, j % nslot, (j + l2) % nslot, (j + l1) % nslot,
                         j + l2 < n, j + l1 < n, j + l1 >= nslot)
            for j in range(n - nslot, n):
                wait_s(j % nslot)
            plsc.subcore_barrier()

            def fire_in(b, t):
                pltpu.async_copy(
                    acc.at[pl.ds(zrow0 + 128 * t, 128)], rows[b], gsem[b])

            def wait_in(b, t):
                pltpu.make_async_copy(
                    acc.at[pl.ds(zrow0 + 128 * t, 128)], rows[b],
                    gsem[b]).wait()

            def fire_out(b, t):
                pltpu.async_copy(
                    rows[b],
                    out.at[pl.ds(zrow0 + 128 * t, 128),
                           pl.ds(cg_id * w, w)], ssem[b])

            def wait_out(b, t):
                pltpu.make_async_copy(
                    rows[b],
                    out.at[pl.ds(zrow0 + 128 * t, 128),
                           pl.ds(cg_id * w, w)], ssem[b]).wait()

            for t in range(zch + 1):
                if t < zch:
                    b = t % nslot
                    if t >= nslot:
                        wait_out(b, t - nslot)
                    fire_in(b, t)
                if t >= 1:
                    b2 = (t - 1) % nslot
                    wait_in(b2, t - 1)
                    fire_out(b2, t - 1)
            for t in range(max(0, zch - nslot), zch):
                wait_out(t % nslot, t)
            if p + 1 < passes:
                plsc.subcore_barrier()

    scratch = [pltpu.VMEM_SHARED((n_acc, w), jnp.float32)]
    scratch += [pltpu.VMEM((ECHUNK,), jnp.int32) for _ in range(nslot)]
    scratch += [pltpu.VMEM((ECHUNK,), jnp.int32) for _ in range(nslot)]
    scratch += [pltpu.VMEM((ECHUNK, w), jnp.float32) for _ in range(nslot)]
    scratch += [pltpu.SemaphoreType.DMA for _ in range(3 * nslot)]
    scratch += [pltpu.SemaphoreType.DMA]

    return pl.kernel(
        body,
        out_type=jax.ShapeDtypeStruct((n_acc, D), jnp.float32),
        mesh=mesh,
        compiler_params=pltpu.CompilerParams(use_tc_tiling_on_sc=False),
        scratch_types=scratch,
    )


def _segsum_sc(table, src2, dst2, n_dst, cg):
    n_src = table.shape[0]
    w = D // cg
    tflat = table.reshape(n_src * cg, w)
    e_pad = src2.shape[0] * ECHUNK
    zeros = jnp.zeros((128, w), jnp.float32)
    kern = _make_segsum(n_src, n_dst, e_pad, cg)
    return kern(tflat, src2, dst2, zeros)


@functools.lru_cache(maxsize=None)
def _make_counts(n_dst, e_pad):
    n_acc = _pad_up(n_dst + 1, NS * 128)
    zch = (n_acc // NS) // 128
    n = e_pad // (NC * NS) // ECHUNK
    mesh = plsc.VectorSubcoreMesh(core_axis_name="c", subcore_axis_name="s",
                                  num_cores=NC, num_subcores=NS)

    def body(dst2, ones_hbm, out, *scr):
        cnt_sh, dstall, onesb, zbuf = scr[0], scr[1], scr[2], scr[3]
        dbuf = scr[4:4 + NSLOT]
        ssem = scr[4 + NSLOT:4 + 2 * NSLOT]
        osem = scr[4 + 2 * NSLOT]

        core = lax.axis_index("c")
        sub = lax.axis_index("s")
        wid = core * NS + sub
        crow0 = wid * n
        zrow0 = sub * (n_acc // NS)

        pltpu.async_copy(dst2.at[pl.ds(crow0, n)], dstall, osem)
        pltpu.async_copy(ones_hbm, onesb, ssem[0])
        pltpu.make_async_copy(dst2.at[pl.ds(crow0, n)], dstall, osem).wait()
        pltpu.make_async_copy(ones_hbm, onesb, ssem[0]).wait()
        zvec = jnp.zeros((16,), jnp.float32)

        def zrow(i, _):
            zbuf[i, pl.ds(0, 16)] = zvec
            return 0
        lax.fori_loop(0, 128, zrow, 0)
        for z in range(zch):
            pltpu.async_copy(zbuf, cnt_sh.at[pl.ds(zrow0 + 128 * z, 128)],
                             osem)
        for z in range(zch):
            pltpu.make_async_copy(
                zbuf, cnt_sh.at[pl.ds(zrow0 + 128 * z, 128)], osem).wait()
        plsc.subcore_barrier()

        def fire_s(b):
            pltpu.async_copy(onesb, cnt_sh.at[dbuf[b]], ssem[b], add=True)

        def wait_s(b):
            pltpu.make_async_copy(onesb, cnt_sh.at[dbuf[b]], ssem[b]).wait()

        for j in range(NSLOT):
            _vcopy_row(dbuf[j], dstall, j)
            fire_s(j)
        n_main = (n - NSLOT) // NSLOT

        def mstep(g, _):
            for b in range(NSLOT):
                j = NSLOT + g * NSLOT + b
                wait_s(b)
                _vcopy_row(dbuf[b], dstall, j)
                fire_s(b)
            return 0
        lax.fori_loop(0, n_main, mstep, 0)
        for j in range(NSLOT + n_main * NSLOT, n):
            b = j % NSLOT
            wait_s(b)
            _vcopy_row(dbuf[b], dstall, j)
            fire_s(b)
        for j in range(n - NSLOT, n):
            wait_s(j % NSLOT)
        plsc.subcore_barrier()

        def ostep(z, _):
            r0 = zrow0 + 128 * z
            pltpu.sync_copy(cnt_sh.at[pl.ds(r0, 128)], zbuf)
            pltpu.sync_copy(zbuf, out.at[core, pl.ds(r0, 128)])
            return 0
        lax.fori_loop(0, zch, ostep, 0)

    scratch = [
        pltpu.VMEM_SHARED((n_acc, 16), jnp.float32),
        pltpu.VMEM((n, ECHUNK), jnp.int32),
        pltpu.VMEM((ECHUNK, 16), jnp.float32),
        pltpu.VMEM((128, 16), jnp.float32),
    ]
    scratch += [pltpu.VMEM((ECHUNK,), jnp.int32) for _ in range(NSLOT)]
    scratch += [pltpu.SemaphoreType.DMA for _ in range(NSLOT)]
    scratch += [pltpu.SemaphoreType.DMA]

    return pl.kernel(
        body,
        out_type=jax.ShapeDtypeStruct((NC, n_acc, 16), jnp.float32),
        mesh=mesh,
        compiler_params=pltpu.CompilerParams(use_tc_tiling_on_sc=False),
        scratch_types=scratch,
    )


def _counts_sc(dst2, n_dst):
    e_pad = dst2.shape[0] * ECHUNK
    ones = jnp.ones((ECHUNK, 16), jnp.float32)
    kern = _make_counts(n_dst, e_pad)
    out = kern(dst2, ones)
    return out[:, :, 0]


def _proj_body(x_ref, w_ref, b_ref, o_ref):
    o_ref[...] = lax.dot_general(
        x_ref[...], w_ref[...], (((1,), (1,)), ((), ())),
        preferred_element_type=jnp.float32) + b_ref[...]


def _proj(x, W, b):
    n = x.shape[0]
    grid = (n + R_BLK - 1) // R_BLK
    return pl.pallas_call(
        _proj_body,
        grid=(grid,),
        in_specs=[
            pl.BlockSpec((R_BLK, D), lambda i: (i, 0)),
            pl.BlockSpec((D, D), lambda i: (0, 0)),
            pl.BlockSpec((1, D), lambda i: (0, 0)),
        ],
        out_specs=pl.BlockSpec((R_BLK, D), lambda i: (i, 0)),
        out_shape=jax.ShapeDtypeStruct((n, D), jnp.float32),
    )(x, W, b.reshape(1, D))


def _sage_body(s_ref, cnt_ref, x_ref, wl_ref, bl_ref, wr_ref, o_ref):
    c = cnt_ref[0, :] + cnt_ref[1, :]
    inv = 1.0 / jnp.maximum(c, 1.0)
    mean = s_ref[...] * inv[:, None]
    o = (lax.dot_general(mean, wl_ref[...], (((1,), (1,)), ((), ())),
                         preferred_element_type=jnp.float32)
         + bl_ref[...]
         + lax.dot_general(x_ref[...], wr_ref[...], (((1,), (1,)), ((), ())),
                           preferred_element_type=jnp.float32))
    o_ref[...] = jnp.maximum(o, 0.0)


def _sage_dense(s_pad, cnt2, x, Wl, bl, Wr):
    n = x.shape[0]
    grid = (n + R_BLK - 1) // R_BLK
    assert cnt2.shape[1] >= grid * R_BLK and s_pad.shape[0] >= grid * R_BLK
    return pl.pallas_call(
        _sage_body,
        grid=(grid,),
        in_specs=[
            pl.BlockSpec((R_BLK, D), lambda i: (i, 0)),
            pl.BlockSpec((2, R_BLK), lambda i: (0, i)),
            pl.BlockSpec((R_BLK, D), lambda i: (i, 0)),
            pl.BlockSpec((D, D), lambda i: (0, 0)),
            pl.BlockSpec((1, D), lambda i: (0, 0)),
            pl.BlockSpec((D, D), lambda i: (0, 0)),
        ],
        out_specs=pl.BlockSpec((R_BLK, D), lambda i: (i, 0)),
        out_shape=jax.ShapeDtypeStruct((n, D), jnp.float32),
    )(s_pad, cnt2, x, Wl, bl.reshape(1, D), Wr)


def _pad_edges(src, dst, n_dst):
    e = src.shape[0]
    e_pad = _pad_up(e, NC * NS * ECHUNK)
    pad = e_pad - e
    src_p = jnp.concatenate([src, jnp.zeros((pad,), jnp.int32)])
    dst_p = jnp.concatenate([dst, jnp.full((pad,), n_dst, jnp.int32)])
    return (src_p.reshape(e_pad // ECHUNK, ECHUNK),
            dst_p.reshape(e_pad // ECHUNK, ECHUNK))


def kernel(venue_x, edge_uv_src, edge_uv_dst, edge_vu_src, edge_vu_dst,
           emb_user, Wp, bp,
           Wl_uv_0, bl_uv_0, Wr_uv_0, Wl_vu_0, bl_vu_0, Wr_vu_0,
           Wl_uv_1, bl_uv_1, Wr_uv_1, Wl_vu_1, bl_vu_1, Wr_vu_1):
    n_user = emb_user.shape[0]
    n_venue = venue_x.shape[0]

    uv_src2, uv_dst2 = _pad_edges(edge_uv_src, edge_uv_dst, n_venue)
    vu_src2, vu_dst2 = _pad_edges(edge_vu_src, edge_vu_dst, n_user)

    cnt_v = _counts_sc(uv_dst2, n_venue)
    cnt_u = _counts_sc(vu_dst2, n_user)

    user = emb_user
    venue = _proj(venue_x, Wp, bp)

    layers = [
        (Wl_uv_0, bl_uv_0, Wr_uv_0, Wl_vu_0, bl_vu_0, Wr_vu_0),
        (Wl_uv_1, bl_uv_1, Wr_uv_1, Wl_vu_1, bl_vu_1, Wr_vu_1),
    ]
    for (Wluv, bluv, Wruv, Wlvu, blvu, Wrvu) in layers:
        s_v = _segsum_sc(user, uv_src2, uv_dst2, n_venue, cg=2)
        s_u = _segsum_sc(venue, vu_src2, vu_dst2, n_user, cg=4)
        venue_new = _sage_dense(s_v, cnt_v, venue, Wluv, bluv, Wruv)
        user_new = _sage_dense(s_u, cnt_u, user, Wlvu, blvu, Wrvu)
        user, venue = user_new, venue_new
    return (user, venue)
